# baseline (speedup 1.0000x reference)
import functools

import jax
import jax.numpy as jnp
from jax import lax
from jax.experimental import pallas as pl
from jax.experimental.pallas import tpu as pltpu
from jax.experimental.pallas import tpu_sc as plsc

_D = 1024
_SEQ = 8192
_B = 4
_NC, _NS, _L = 2, 16, 16
_NW = _NC * _NS
_RW = _SEQ // _NW
_R = 16
_J = _RW // _R
_T = _J * _B
_NBUF = 5

_mesh = plsc.VectorSubcoreMesh(core_axis_name="c", subcore_axis_name="s")


@functools.partial(
    pl.kernel,
    out_type=jax.ShapeDtypeStruct((_B * _SEQ, _D), jnp.float32),
    mesh=_mesh,
    scratch_types=(
        [pltpu.VMEM((_R, _D), jnp.float32) for _ in range(2 + _NBUF)]
        + [pltpu.SemaphoreType.DMA for _ in range(2 + _NBUF)]
    ),
)
def _pos_add(x_hbm, pos_hbm, out_hbm, *scratch):
    p_bufs = list(scratch[0:2])
    x_bufs = list(scratch[2:2 + _NBUF])
    p_sems = list(scratch[2 + _NBUF:4 + _NBUF])
    x_sems = list(scratch[4 + _NBUF:4 + 2 * _NBUF])
    wid = lax.axis_index("s") * _NC + lax.axis_index("c")
    base = wid * _RW

    def pos_copy(j):
        return pltpu.async_copy(
            pos_hbm.at[pl.ds(base + j * _R, _R), :], p_bufs[j % 2], p_sems[j % 2])

    def x_row(t):
        j, b = divmod(t, _B)
        return b * _SEQ + base + j * _R

    def x_in(t):
        return pltpu.async_copy(
            x_hbm.at[pl.ds(x_row(t), _R), :], x_bufs[t % _NBUF], x_sems[t % _NBUF])

    def x_out(t):
        return pltpu.async_copy(
            x_bufs[t % _NBUF], out_hbm.at[pl.ds(x_row(t), _R), :], x_sems[t % _NBUF])

    ind, outd, pd = {}, {}, {}
    pd[0] = pos_copy(0)
    for k in range(_NBUF):
        ind[k] = x_in(k)
    for t in range(_T):
        j, b = divmod(t, _B)
        if b == 0:
            pd[j].wait()
            if j + 1 < _J:
                pd[j + 1] = pos_copy(j + 1)
        ind[t].wait()
        if t >= 2 and t - 2 + _NBUF < _T:
            outd[t - 2].wait()
            ind[t - 2 + _NBUF] = x_in(t - 2 + _NBUF)
        p_buf, x_buf = p_bufs[j % 2], x_bufs[t % _NBUF]

        @plsc.parallel_loop(0, _R * _D, step=_L, unroll=16)
        def _add(i):
            r = i >> 10
            c = pl.multiple_of(i & (_D - 1), _L)
            x_buf[r, pl.ds(c, _L)] = x_buf[r, pl.ds(c, _L)] + p_buf[r, pl.ds(c, _L)]

        outd[t] = x_out(t)
    for t in range(_T - _NBUF, _T):
        outd[t].wait()


def kernel(x, pos_embedding):
    out = _pos_add(x.reshape(_B * _SEQ, _D), pos_embedding)
    return out.reshape(x.shape)


# speedup vs baseline: 1.0297x; 1.0297x over previous
import functools

import jax
import jax.numpy as jnp
from jax import lax
from jax.experimental import pallas as pl
from jax.experimental.pallas import tpu as pltpu
from jax.experimental.pallas import tpu_sc as plsc

_D = 1024
_SEQ = 8192
_B = 4
_NC, _NS, _L = 2, 16, 16
_NW = _NC * _NS
_RW = _SEQ // _NW
_R = 16
_J = _RW // _R
_T = _J * _B
_NBUF = 5

_mesh = plsc.VectorSubcoreMesh(core_axis_name="c", subcore_axis_name="s")


@functools.partial(
    pl.kernel,
    out_type=jax.ShapeDtypeStruct((_B * _SEQ, _D), jnp.float32),
    mesh=_mesh,
    scratch_types=(
        [pltpu.VMEM((_R, _D), jnp.float32) for _ in range(2 + _NBUF)]
        + [pltpu.SemaphoreType.DMA for _ in range(2 + _NBUF)]
    ),
)
def _pos_add(x_hbm, pos_hbm, out_hbm, *scratch):
    p_bufs = list(scratch[0:2])
    x_bufs = list(scratch[2:2 + _NBUF])
    p_sems = list(scratch[2 + _NBUF:4 + _NBUF])
    x_sems = list(scratch[4 + _NBUF:4 + 2 * _NBUF])
    wid = lax.axis_index("s") * _NC + lax.axis_index("c")
    base = wid * _RW

    def pos_copy(j):
        return pltpu.async_copy(
            pos_hbm.at[pl.ds(base + j * _R, _R), :], p_bufs[j % 2], p_sems[j % 2])

    def x_row(t):
        j, b = divmod(t, _B)
        return b * _SEQ + base + j * _R

    def x_in(t):
        return pltpu.async_copy(
            x_hbm.at[pl.ds(x_row(t), _R), :], x_bufs[t % _NBUF], x_sems[t % _NBUF])

    def x_out(t):
        return pltpu.async_copy(
            x_bufs[t % _NBUF], out_hbm.at[pl.ds(x_row(t), _R), :], x_sems[t % _NBUF])

    ind, outd, pd = {}, {}, {}
    pd[0] = pos_copy(0)
    for k in range(_NBUF):
        ind[k] = x_in(k)
    for t in range(_T):
        j, b = divmod(t, _B)
        if b == 0:
            pd[j].wait()
            if j + 1 < _J:
                pd[j + 1] = pos_copy(j + 1)
        ind[t].wait()
        if t >= 2 and t - 2 + _NBUF < _T:
            outd[t - 2].wait()
            ind[t - 2 + _NBUF] = x_in(t - 2 + _NBUF)
        p_buf, x_buf = p_bufs[j % 2], x_bufs[t % _NBUF]

        @plsc.parallel_loop(0, _R * _D, step=_L, unroll=8)
        def _add(i):
            r = i >> 10
            c = pl.multiple_of(i & (_D - 1), _L)
            plsc.addupdate(x_buf.at[r, pl.ds(c, _L)], p_buf[r, pl.ds(c, _L)])

        outd[t] = x_out(t)
    for t in range(_T - _NBUF, _T):
        outd[t].wait()


def kernel(x, pos_embedding):
    out = _pos_add(x.reshape(_B * _SEQ, _D), pos_embedding)
    return out.reshape(x.shape)
